# baseline (speedup 1.0000x reference)
import jax
import jax.numpy as jnp
from jax import lax
from jax.experimental import pallas as pl
from jax.experimental.pallas import tpu as pltpu
from jax.experimental.pallas import tpu_sc as plsc

_B, _N, _D, _T = 16, 192, 78, 6
_G = _N // 16
_Q = 4
_BIGF = 3.0e38


def _cost_iou_body(imgw_ref, pred_ref, tgt_ref, out_ref):
    w = imgw_ref[0, 0].astype(jnp.float32)
    for b in range(_B):
        _cost_iou_one(w, pred_ref[b], tgt_ref[b], out_ref, b)


def _cost_iou_one(w, pred, tgt, out_ref, b):
    predT = pred.T
    tgtT = tgt.T
    tcoord = tgtT[6:, :]
    invalT = (tcoord < 0.0) | (tcoord >= w)
    nv6 = jnp.sum(jnp.where(invalT, 0.0, 1.0), axis=0, keepdims=True)
    dist_rows, iou_rows = [], []
    for t in range(_T):
        acc = jnp.zeros((8, _N), jnp.float32)
        for c0 in range(0, _D - 6, 8):
            pc = predT[6 + c0:14 + c0, :]
            tc = tgtT[6 + c0:14 + c0, t:t + 1]
            inval = (tc < 0.0) | (tc >= w)
            acc = acc + jnp.where(inval, 0.0, jnp.abs(pc - tc))
        s1 = jnp.sum(acc, axis=0, keepdims=True)
        nv = nv6[0:1, t:t + 1]
        dist_rows.append(s1 / (jnp.maximum(nv, 1.0) + 1e-6))
        iou_rows.append((30.0 * nv - s1) / (30.0 * nv + s1 + 1e-9))
    dist = jnp.concatenate(dist_rows, axis=0)
    iou = jnp.maximum(jnp.concatenate(iou_rows, axis=0), 0.0)
    sdx = predT[2:3, :] - tgt[:, 2:3]
    sdy = predT[3:4, :] - tgt[:, 3:4]
    sd = jnp.sqrt(sdx * sdx + sdy * sdy)
    th = jnp.abs(predT[4:5, :] - tgt[:, 4:5])
    dsc = 1.0 - dist / jnp.maximum(jnp.max(dist), 1e-6) + 0.01
    ssc = 1.0 - sd / jnp.maximum(jnp.max(sd), 1e-6) + 0.01
    tsc = 1.0 - th / jnp.maximum(jnp.max(th), 1e-6) + 0.01
    logits = predT[0:2, :]
    pr = 1.0 / (1.0 + jnp.exp(-logits))
    negc = -jnp.log(1.0 - pr + 1e-12) * 0.75 * (pr * pr)
    posc = -jnp.log(pr + 1e-12) * 0.25 * ((1.0 - pr) * (1.0 - pr))
    colcost = posc - negc
    cls = jnp.where(tgt[:, 1:2] >= 1.0, colcost[1:2, :], colcost[0:1, :])
    score = dsc * ssc * tsc
    out_ref[b, 0:_T] = -(score * score) * 3.0 + cls
    out_ref[b, _T:2 * _T] = iou


def _assign_body(ci_hbm, out_hbm, ci_v, out_v, tmpf, tmpi, shf, shi):
    cid = lax.axis_index("c")
    sid = lax.axis_index("s")
    img = cid * 8 + sid // 2
    h = sid % 2

    pltpu.sync_copy(ci_hbm.at[img], ci_v)
    lanes = lax.iota(jnp.int32, 16)
    zero_i = jnp.zeros((16,), jnp.int32)
    flatn = [lanes + g * 16 for g in range(_G)]

    def splat(v):
        return jnp.zeros((16,), v.dtype) + v

    def extremum(work, is_min):
        m = work[0]
        for g in range(1, _G):
            m = jnp.minimum(m, work[g]) if is_min else \
                jnp.maximum(m, work[g])
        mm = splat(jnp.min(m) if is_min else jnp.max(m))
        selg = jnp.full((16,), 127, jnp.int32)
        sell = zero_i
        for g in range(_G - 1, -1, -1):
            eq = work[g] == mm
            has = plsc.all_reduce_population_count(eq) > 0
            ffs = plsc.all_reduce_ffs(eq)
            selg = jnp.where(has, g, selg)
            sell = jnp.where(has, ffs, sell)
        nsel = selg * 16 + sell
        return mm, nsel

    def per_target(ti, carry):
        cnt, first, bestc, bestt = carry
        t = h * 3 + ti
        cg = [ci_v[pl.ds(t * _N + g * 16, 16)] for g in range(_G)]
        ig = [ci_v[pl.ds(_T * _N + t * _N + g * 16, 16)]
              for g in range(_G)]
        newbc, newbt = [], []
        for g in range(_G):
            lt = cg[g] < bestc[g]
            newbc.append(jnp.where(lt, cg[g], bestc[g]))
            newbt.append(jnp.where(lt, t, bestt[g]))
        work = list(ig)
        ssum = jnp.zeros((16,), jnp.float32)
        for _ in range(_Q):
            mx, nsel = extremum(work, is_min=False)
            ssum = ssum + mx
            for g in range(_G):
                work[g] = jnp.where(flatn[g] == nsel, -_BIGF, work[g])
        kt = jnp.maximum(ssum.astype(jnp.int32), 1)
        workc = list(cg)
        newcnt = list(cnt)
        newfirst = list(first)
        for j in range(_Q):
            mn, nsel = extremum(workc, is_min=True)
            active = j < kt
            for g in range(_G):
                hit = flatn[g] == nsel
                workc[g] = jnp.where(hit, _BIGF, workc[g])
                take = hit & active
                newcnt[g] = newcnt[g] + take.astype(jnp.int32)
                newfirst[g] = jnp.where(take & (newfirst[g] >= 99), t,
                                        newfirst[g])
        return (tuple(newcnt), tuple(newfirst),
                tuple(newbc), tuple(newbt))

    init = (tuple(zero_i for _ in range(_G)),
            tuple(jnp.full((16,), 99, jnp.int32) for _ in range(_G)),
            tuple(jnp.full((16,), _BIGF, jnp.float32) for _ in range(_G)),
            tuple(zero_i for _ in range(_G)))
    cnt, first, bestc, bestt = lax.fori_loop(0, _T // 2, per_target, init)

    @pl.when(h == 1)
    def _():
        for g in range(_G):
            tmpi[pl.ds(g * 16, 16)] = cnt[g]
            tmpi[pl.ds(_N + g * 16, 16)] = first[g]
            tmpi[pl.ds(2 * _N + g * 16, 16)] = bestt[g]
            tmpf[pl.ds(g * 16, 16)] = bestc[g]
        pltpu.sync_copy(tmpi, shi.at[sid])
        pltpu.sync_copy(tmpf, shf.at[sid])

    plsc.subcore_barrier()

    @pl.when(h == 0)
    def _():
        pltpu.sync_copy(shi.at[sid + 1], tmpi)
        pltpu.sync_copy(shf.at[sid + 1], tmpf)
        for g in range(_G):
            pc = tmpi[pl.ds(g * 16, 16)]
            pf = tmpi[pl.ds(_N + g * 16, 16)]
            pt = tmpi[pl.ds(2 * _N + g * 16, 16)]
            pb = tmpf[pl.ds(g * 16, 16)]
            tot = cnt[g] + pc
            fmin = jnp.minimum(first[g], pf)
            bt = jnp.where(pb < bestc[g], pt, bestt[g])
            gt = jnp.where(tot > 1, bt, fmin)
            out_v[pl.ds(g * 16, 16)] = jnp.where(tot > 0, gt, -1)
        pltpu.sync_copy(out_v, out_hbm.at[img])


def kernel(preds, targets, masks, img_w, img_h):
    del masks, img_h
    imgw = jnp.reshape(jnp.asarray(img_w, jnp.int32), (1, 1))
    ci = pl.pallas_call(
        _cost_iou_body,
        out_shape=jax.ShapeDtypeStruct((_B, 2 * _T, _N), jnp.float32),
    )(imgw, preds, targets)
    matched = pl.kernel(
        _assign_body,
        out_type=jax.ShapeDtypeStruct((_B, _N), jnp.int32),
        mesh=plsc.VectorSubcoreMesh(core_axis_name="c", subcore_axis_name="s"),
        compiler_params=pltpu.CompilerParams(needs_layout_passes=False),
        scratch_types=[pltpu.VMEM((2 * _N * _T,), jnp.float32),
                       pltpu.VMEM((_N,), jnp.int32),
                       pltpu.VMEM((_N,), jnp.float32),
                       pltpu.VMEM((3 * _N,), jnp.int32),
                       pltpu.VMEM_SHARED((16, _N), jnp.float32),
                       pltpu.VMEM_SHARED((16, 3 * _N), jnp.int32)],
    )(ci.reshape(_B, 2 * _N * _T))
    return matched >= 0, matched


# speedup vs baseline: 1.0056x; 1.0056x over previous
import jax
import jax.numpy as jnp
from jax import lax
from jax.experimental import pallas as pl
from jax.experimental.pallas import tpu as pltpu
from jax.experimental.pallas import tpu_sc as plsc

_B, _N, _D, _T = 16, 192, 78, 6
_G = _N // 16
_Q = 4
_BIGF = 3.0e38


def _cost_iou_body(imgw_ref, pred_ref, tgt_ref, out_ref):
    w = imgw_ref[0, 0].astype(jnp.float32)
    for b in range(_B):
        _cost_iou_one(w, pred_ref[b], tgt_ref[b], out_ref, b)


def _cost_iou_one(w, pred, tgt, out_ref, b):
    predT = pred.T
    tgtT = tgt.T
    tcoord = tgtT[6:, :]
    invalT = (tcoord < 0.0) | (tcoord >= w)
    nv6 = jnp.sum(jnp.where(invalT, 0.0, 1.0), axis=0, keepdims=True)
    dist_rows, iou_rows = [], []
    for t in range(_T):
        acc = jnp.zeros((8, _N), jnp.float32)
        for c0 in range(0, _D - 6, 8):
            pc = predT[6 + c0:14 + c0, :]
            tc = tgtT[6 + c0:14 + c0, t:t + 1]
            inval = (tc < 0.0) | (tc >= w)
            acc = acc + jnp.where(inval, 0.0, jnp.abs(pc - tc))
        s1 = jnp.sum(acc, axis=0, keepdims=True)
        nv = nv6[0:1, t:t + 1]
        dist_rows.append(s1 / (jnp.maximum(nv, 1.0) + 1e-6))
        iou_rows.append((30.0 * nv - s1) / (30.0 * nv + s1 + 1e-9))
    dist = jnp.concatenate(dist_rows, axis=0)
    iou = jnp.maximum(jnp.concatenate(iou_rows, axis=0), 0.0)
    sdx = predT[2:3, :] - tgt[:, 2:3]
    sdy = predT[3:4, :] - tgt[:, 3:4]
    sd = jnp.sqrt(sdx * sdx + sdy * sdy)
    th = jnp.abs(predT[4:5, :] - tgt[:, 4:5])
    dsc = 1.0 - dist / jnp.maximum(jnp.max(dist), 1e-6) + 0.01
    ssc = 1.0 - sd / jnp.maximum(jnp.max(sd), 1e-6) + 0.01
    tsc = 1.0 - th / jnp.maximum(jnp.max(th), 1e-6) + 0.01
    logits = predT[0:2, :]
    pr = 1.0 / (1.0 + jnp.exp(-logits))
    negc = -jnp.log(1.0 - pr + 1e-12) * 0.75 * (pr * pr)
    posc = -jnp.log(pr + 1e-12) * 0.25 * ((1.0 - pr) * (1.0 - pr))
    colcost = posc - negc
    cls = jnp.where(tgt[:, 1:2] >= 1.0, colcost[1:2, :], colcost[0:1, :])
    score = dsc * ssc * tsc
    out_ref[b, 0:_T] = -(score * score) * 3.0 + cls
    out_ref[b, _T:2 * _T] = iou


def _assign_body(ci_hbm, out_hbm, ci_v, out_v):
    cid = lax.axis_index("c")
    sid = lax.axis_index("s")
    img = cid * 8 + sid

    @pl.when(sid < 8)
    def _():
        pltpu.sync_copy(ci_hbm.at[img], ci_v)
        lanes = lax.iota(jnp.int32, 16)
        zero_i = jnp.zeros((16,), jnp.int32)
        flatn = [lanes + g * 16 for g in range(_G)]

        def splat(v):
            return jnp.zeros((16,), v.dtype) + v

        def extremum(work, is_min):
            m = work[0]
            for g in range(1, _G):
                m = jnp.minimum(m, work[g]) if is_min else \
                    jnp.maximum(m, work[g])
            mm = splat(jnp.min(m) if is_min else jnp.max(m))
            selg = jnp.full((16,), 127, jnp.int32)
            sell = zero_i
            for g in range(_G - 1, -1, -1):
                eq = work[g] == mm
                has = plsc.all_reduce_population_count(eq) > 0
                ffs = plsc.all_reduce_ffs(eq)
                selg = jnp.where(has, g, selg)
                sell = jnp.where(has, ffs, sell)
            nsel = selg * 16 + sell
            return mm, nsel

        def per_target(t, carry):
            cnt, first, bestc, bestt = carry
            cg = [ci_v[pl.ds(t * _N + g * 16, 16)] for g in range(_G)]
            ig = [ci_v[pl.ds(_T * _N + t * _N + g * 16, 16)]
                  for g in range(_G)]
            newbc, newbt = [], []
            for g in range(_G):
                lt = cg[g] < bestc[g]
                newbc.append(jnp.where(lt, cg[g], bestc[g]))
                newbt.append(jnp.where(lt, t, bestt[g]))
            work = list(ig)
            ssum = jnp.zeros((16,), jnp.float32)
            for _ in range(_Q):
                mx, nsel = extremum(work, is_min=False)
                ssum = ssum + mx
                for g in range(_G):
                    work[g] = jnp.where(flatn[g] == nsel, -_BIGF, work[g])
            kt = jnp.maximum(ssum.astype(jnp.int32), 1)
            workc = list(cg)
            newcnt = list(cnt)
            newfirst = list(first)
            for j in range(_Q):
                mn, nsel = extremum(workc, is_min=True)
                active = j < kt
                for g in range(_G):
                    hit = flatn[g] == nsel
                    workc[g] = jnp.where(hit, _BIGF, workc[g])
                    take = hit & active
                    newcnt[g] = newcnt[g] + take.astype(jnp.int32)
                    newfirst[g] = jnp.where(take & (newfirst[g] >= 99), t,
                                            newfirst[g])
            return (tuple(newcnt), tuple(newfirst),
                    tuple(newbc), tuple(newbt))

        init = (tuple(zero_i for _ in range(_G)),
                tuple(jnp.full((16,), 99, jnp.int32) for _ in range(_G)),
                tuple(jnp.full((16,), _BIGF, jnp.float32) for _ in range(_G)),
                tuple(zero_i for _ in range(_G)))
        cnt, first, bestc, bestt = lax.fori_loop(0, _T, per_target, init)
        for g in range(_G):
            gt = jnp.where(cnt[g] > 1, bestt[g], first[g])
            out_v[pl.ds(g * 16, 16)] = jnp.where(cnt[g] > 0, gt, -1)
        pltpu.sync_copy(out_v, out_hbm.at[img])


def kernel(preds, targets, masks, img_w, img_h):
    del masks, img_h
    imgw = jnp.reshape(jnp.asarray(img_w, jnp.int32), (1, 1))
    ci = pl.pallas_call(
        _cost_iou_body,
        out_shape=jax.ShapeDtypeStruct((_B, 2 * _T, _N), jnp.float32),
    )(imgw, preds, targets)
    matched = pl.kernel(
        _assign_body,
        out_type=jax.ShapeDtypeStruct((_B, _N), jnp.int32),
        mesh=plsc.VectorSubcoreMesh(core_axis_name="c", subcore_axis_name="s"),
        compiler_params=pltpu.CompilerParams(needs_layout_passes=False),
        scratch_types=[pltpu.VMEM((2 * _N * _T,), jnp.float32),
                       pltpu.VMEM((_N,), jnp.int32)],
    )(ci.reshape(_B, 2 * _N * _T))
    return matched >= 0, matched


# speedup vs baseline: 1.0545x; 1.0487x over previous
import jax
import jax.numpy as jnp
from jax import lax
from jax.experimental import pallas as pl
from jax.experimental.pallas import tpu as pltpu
from jax.experimental.pallas import tpu_sc as plsc

_B, _N, _D, _T = 16, 192, 78, 6
_G = _N // 16
_Q = 4
_BIGF = 3.0e38


def _cost_iou_body(imgw_ref, pred_ref, tgt_ref, out_ref):
    w = imgw_ref[0, 0].astype(jnp.float32)
    for b in range(_B):
        _cost_iou_one(w, pred_ref[b], tgt_ref[b], out_ref, b)


def _cost_iou_one(w, pred, tgt, out_ref, b):
    predT = pred.T
    tgtT = tgt.T
    tcoord = tgtT[6:, :]
    invalT = (tcoord < 0.0) | (tcoord >= w)
    nv6 = jnp.sum(jnp.where(invalT, 0.0, 1.0), axis=0, keepdims=True)
    dist_rows, iou_rows = [], []
    for t in range(_T):
        acc = jnp.zeros((8, _N), jnp.float32)
        for c0 in range(0, _D - 6, 8):
            pc = predT[6 + c0:14 + c0, :]
            tc = tgtT[6 + c0:14 + c0, t:t + 1]
            inval = (tc < 0.0) | (tc >= w)
            acc = acc + jnp.where(inval, 0.0, jnp.abs(pc - tc))
        s1 = jnp.sum(acc, axis=0, keepdims=True)
        nv = nv6[0:1, t:t + 1]
        dist_rows.append(s1 / (jnp.maximum(nv, 1.0) + 1e-6))
        iou_rows.append((30.0 * nv - s1) / (30.0 * nv + s1 + 1e-9))
    dist = jnp.concatenate(dist_rows, axis=0)
    iou = jnp.maximum(jnp.concatenate(iou_rows, axis=0), 0.0)
    sdx = predT[2:3, :] - tgt[:, 2:3]
    sdy = predT[3:4, :] - tgt[:, 3:4]
    sd = jnp.sqrt(sdx * sdx + sdy * sdy)
    th = jnp.abs(predT[4:5, :] - tgt[:, 4:5])
    dsc = 1.0 - dist / jnp.maximum(jnp.max(dist), 1e-6) + 0.01
    ssc = 1.0 - sd / jnp.maximum(jnp.max(sd), 1e-6) + 0.01
    tsc = 1.0 - th / jnp.maximum(jnp.max(th), 1e-6) + 0.01
    logits = predT[0:2, :]
    pr = 1.0 / (1.0 + jnp.exp(-logits))
    negc = -jnp.log(1.0 - pr + 1e-12) * 0.75 * (pr * pr)
    posc = -jnp.log(pr + 1e-12) * 0.25 * ((1.0 - pr) * (1.0 - pr))
    colcost = posc - negc
    cls = jnp.where(tgt[:, 1:2] >= 1.0, colcost[1:2, :], colcost[0:1, :])
    score = dsc * ssc * tsc
    out_ref[b, 0:_T] = -(score * score) * 3.0 + cls
    out_ref[b, _T:2 * _T] = iou


def _assign_body(ci_hbm, out_hbm, ci_v, out_v):
    cid = lax.axis_index("c")
    sid = lax.axis_index("s")
    img = cid * 8 + sid

    @pl.when(sid < 8)
    def _():
        pltpu.sync_copy(ci_hbm.at[img], ci_v)
        lanes = lax.iota(jnp.int32, 16)
        zero_i = jnp.zeros((16,), jnp.int32)
        flatn = [lanes + g * 16 for g in range(_G)]

        def splat(v):
            return jnp.zeros((16,), v.dtype) + v

        def extremum(work, is_min):
            m = work[0]
            for g in range(1, _G):
                m = jnp.minimum(m, work[g]) if is_min else \
                    jnp.maximum(m, work[g])
            mm = splat(jnp.min(m) if is_min else jnp.max(m))
            selg = jnp.full((16,), 127, jnp.int32)
            sell = zero_i
            for g in range(_G - 1, -1, -1):
                eq = work[g] == mm
                has = plsc.all_reduce_population_count(eq) > 0
                ffs = plsc.all_reduce_ffs(eq)
                selg = jnp.where(has, g, selg)
                sell = jnp.where(has, ffs, sell)
            nsel = selg * 16 + sell
            return mm, nsel

        def per_target(t, carry):
            cnt, first, bestc, bestt = carry
            cg = [ci_v[t, pl.ds(g * 16, 16)] for g in range(_G)]
            ig = [ci_v[_T + t, pl.ds(g * 16, 16)] for g in range(_G)]
            newbc, newbt = [], []
            for g in range(_G):
                lt = cg[g] < bestc[g]
                newbc.append(jnp.where(lt, cg[g], bestc[g]))
                newbt.append(jnp.where(lt, t, bestt[g]))
            work = list(ig)
            ssum = jnp.zeros((16,), jnp.float32)
            for _ in range(_Q):
                mx, nsel = extremum(work, is_min=False)
                ssum = ssum + mx
                for g in range(_G):
                    work[g] = jnp.where(flatn[g] == nsel, -_BIGF, work[g])
            kt = jnp.maximum(ssum.astype(jnp.int32), 1)
            workc = list(cg)
            newcnt = list(cnt)
            newfirst = list(first)
            for j in range(_Q):
                mn, nsel = extremum(workc, is_min=True)
                active = j < kt
                for g in range(_G):
                    hit = flatn[g] == nsel
                    workc[g] = jnp.where(hit, _BIGF, workc[g])
                    take = hit & active
                    newcnt[g] = newcnt[g] + take.astype(jnp.int32)
                    newfirst[g] = jnp.where(take & (newfirst[g] >= 99), t,
                                            newfirst[g])
            return (tuple(newcnt), tuple(newfirst),
                    tuple(newbc), tuple(newbt))

        init = (tuple(zero_i for _ in range(_G)),
                tuple(jnp.full((16,), 99, jnp.int32) for _ in range(_G)),
                tuple(jnp.full((16,), _BIGF, jnp.float32) for _ in range(_G)),
                tuple(zero_i for _ in range(_G)))
        cnt, first, bestc, bestt = lax.fori_loop(0, _T, per_target, init)
        for g in range(_G):
            gt = jnp.where(cnt[g] > 1, bestt[g], first[g])
            out_v[pl.ds(g * 16, 16)] = jnp.where(cnt[g] > 0, gt, -1)
        pltpu.sync_copy(out_v, out_hbm.at[img])


def kernel(preds, targets, masks, img_w, img_h):
    del masks, img_h
    imgw = jnp.reshape(jnp.asarray(img_w, jnp.int32), (1, 1))
    ci = pl.pallas_call(
        _cost_iou_body,
        out_shape=jax.ShapeDtypeStruct((_B, 2 * _T, _N), jnp.float32),
    )(imgw, preds, targets)
    matched = pl.kernel(
        _assign_body,
        out_type=jax.ShapeDtypeStruct((_B, _N), jnp.int32),
        mesh=plsc.VectorSubcoreMesh(core_axis_name="c", subcore_axis_name="s"),
        compiler_params=pltpu.CompilerParams(needs_layout_passes=False),
        scratch_types=[pltpu.VMEM((2 * _T, _N), jnp.float32),
                       pltpu.VMEM((_N,), jnp.int32)],
    )(ci)
    return matched >= 0, matched
